# baseline (speedup 1.0000x reference)
import functools

import jax
import jax.numpy as jnp
from jax import lax
from jax.experimental import pallas as pl
from jax.experimental.pallas import tpu as pltpu
from jax.experimental.pallas import tpu_sc as plsc

N = 50000
D = 64
H = 64
G = 128
HALF = 32
NC = 2
NS = 16
LANES = 128
K = 4
SUPER = 4
KD = 8
DW = 8
N_PAD = 50048
ROWS_PER_TILE = N_PAD // NS
BLK = 3128
NBLK = N_PAD // BLK


def _feat_body(xcat, src2, dstr, z32, agg2,
               srcQ, dstQ, rows_v, acc_sh, isem, gsem, ssem):
    c = lax.axis_index("c")
    s = lax.axis_index("s")
    rows_e_tile = src2.shape[1] // NS
    n_super = rows_e_tile // (K * SUPER)
    base = s * rows_e_tile

    r0 = s * ROWS_PER_TILE
    pltpu.sync_copy(z32.at[pl.ds(r0, ROWS_PER_TILE)],
                    acc_sh.at[pl.ds(r0, ROWS_PER_TILE)])
    plsc.subcore_barrier()

    def superchunk(q, carry):
        row0 = base + q * (K * SUPER)
        pltpu.async_copy(src2.at[c, pl.ds(row0, K * SUPER)], srcQ, isem)
        pltpu.async_copy(dstr.at[pl.ds(row0, K * SUPER)], dstQ, isem)
        pltpu.make_async_copy(
            src2.at[c, pl.ds(base, K * SUPER)], srcQ, isem).wait()
        pltpu.make_async_copy(
            dstr.at[pl.ds(base, K * SUPER)], dstQ, isem).wait()
        for t in range(SUPER):
            descs = []
            for j in range(K):
                descs.append(pltpu.async_copy(
                    xcat.at[srcQ.at[t * K + j]],
                    rows_v.at[pl.ds(j * LANES, LANES)], gsem))
            for d_ in descs:
                d_.wait()
            descs = []
            for j in range(K):
                descs.append(pltpu.async_copy(
                    rows_v.at[pl.ds(j * LANES, LANES)],
                    acc_sh.at[dstQ.at[t * K + j]], ssem, add=True))
            for d_ in descs:
                d_.wait()
        return carry

    lax.fori_loop(0, n_super, superchunk, 0)
    plsc.subcore_barrier()

    pltpu.sync_copy(acc_sh.at[pl.ds(r0, ROWS_PER_TILE)],
                    agg2.at[c, pl.ds(r0, ROWS_PER_TILE)])


def _deg_body(dst3, zdeg, ones8, dego,
              dst_v, ones_v, deg_sh, dsem):
    c = lax.axis_index("c")
    s = lax.axis_index("s")
    rows_tile = dst3.shape[1] // NS
    n_chunks = rows_tile // KD

    pltpu.sync_copy(ones8, ones_v)
    r0 = s * ROWS_PER_TILE
    pltpu.sync_copy(zdeg.at[pl.ds(r0, ROWS_PER_TILE)],
                    deg_sh.at[pl.ds(r0, ROWS_PER_TILE)])
    plsc.subcore_barrier()

    def chunk(g, carry):
        row0 = s * rows_tile + g * KD
        pltpu.sync_copy(dst3.at[c, pl.ds(row0, KD)], dst_v)
        descs = []
        for j in range(KD):
            descs.append(pltpu.async_copy(
                ones_v, deg_sh.at[dst_v.at[j]], dsem, add=True))
        for d_ in descs:
            d_.wait()
        return carry

    lax.fori_loop(0, n_chunks, chunk, 0)
    plsc.subcore_barrier()

    pltpu.sync_copy(deg_sh.at[pl.ds(r0, ROWS_PER_TILE)],
                    dego.at[c, pl.ds(r0, ROWS_PER_TILE)])


def _sc_aggregate(xcat, src2, dstr, dst3):
    mesh = plsc.VectorSubcoreMesh(core_axis_name="c", subcore_axis_name="s")
    z32 = jnp.zeros((N_PAD, HALF), jnp.float32)
    feat = pl.kernel(
        _feat_body,
        out_type=jax.ShapeDtypeStruct((NC, N_PAD, HALF), jnp.float32),
        mesh=mesh,
        compiler_params=pltpu.CompilerParams(use_tc_tiling_on_sc=False),
        scratch_types=[
            pltpu.VMEM((SUPER * K, LANES), jnp.int32),
            pltpu.VMEM((SUPER * K, LANES), jnp.int32),
            pltpu.VMEM((K * LANES, HALF), jnp.float32),
            pltpu.VMEM_SHARED((N_PAD, HALF), jnp.float32),
            pltpu.SemaphoreType.DMA,
            pltpu.SemaphoreType.DMA,
            pltpu.SemaphoreType.DMA,
        ],
    )
    agg2 = feat(xcat, src2, dstr, z32)

    zdeg = jnp.zeros((N_PAD, DW), jnp.float32)
    ones8 = jnp.ones((LANES, DW), jnp.float32)
    deg = pl.kernel(
        _deg_body,
        out_type=jax.ShapeDtypeStruct((NC, N_PAD, DW), jnp.float32),
        mesh=mesh,
        compiler_params=pltpu.CompilerParams(use_tc_tiling_on_sc=False),
        scratch_types=[
            pltpu.VMEM((KD, LANES), jnp.int32),
            pltpu.VMEM((LANES, DW), jnp.float32),
            pltpu.VMEM_SHARED((N_PAD, DW), jnp.float32),
            pltpu.SemaphoreType.DMA,
        ],
    )
    dego = deg(dst3, zdeg, ones8)
    return agg2, dego


def _tc_body(a2_ref, deg_ref, batch_ref, wg_ref, bg_ref, w1_ref, b1_ref,
             w2_ref, b2_ref, out_ref, g_acc, c_acc):
    i = pl.program_id(0)
    a = a2_ref[...]
    d = deg_ref[...]
    deg = d[0, :, 0:1] + d[1, :, 0:1]
    inv = 1.0 / jnp.maximum(deg, 1.0)
    n0 = a[0] * inv
    n1 = a[1] * inv
    h = n0 @ wg_ref[0:HALF, :] + n1 @ wg_ref[HALF:D, :] + bg_ref[...]
    h = jnp.maximum(h, 0.0)
    b_ids = batch_ref[...]
    oh = (b_ids == lax.broadcasted_iota(jnp.int32, (BLK, G), 1))
    oh = oh.astype(jnp.float32)
    g_part = lax.dot_general(oh, h, (((0,), (0,)), ((), ())))
    c_part = lax.dot_general(oh, jnp.ones((BLK, 1), jnp.float32),
                             (((0,), (0,)), ((), ())))

    @pl.when(i == 0)
    def _():
        g_acc[...] = jnp.zeros_like(g_acc)
        c_acc[...] = jnp.zeros_like(c_acc)

    g_acc[...] += g_part
    c_acc[...] += c_part

    @pl.when(i == NBLK - 1)
    def _():
        gm = g_acc[...] / jnp.maximum(c_acc[...], 1.0)
        t = jnp.maximum(gm @ w1_ref[...] + b1_ref[...], 0.0)
        out_ref[...] = t @ w2_ref[...] + b2_ref[...]


def _tc_head(agg2, dego, batchp, Wg, bg, W1, b1, W2, b2):
    return pl.pallas_call(
        _tc_body,
        grid=(NBLK,),
        in_specs=[
            pl.BlockSpec((NC, BLK, HALF), lambda i: (0, i, 0)),
            pl.BlockSpec((NC, BLK, DW), lambda i: (0, i, 0)),
            pl.BlockSpec((BLK, 1), lambda i: (i, 0)),
            pl.BlockSpec((D, H), lambda i: (0, 0)),
            pl.BlockSpec((1, H), lambda i: (0, 0)),
            pl.BlockSpec((H, H), lambda i: (0, 0)),
            pl.BlockSpec((1, H), lambda i: (0, 0)),
            pl.BlockSpec((H, H), lambda i: (0, 0)),
            pl.BlockSpec((1, H), lambda i: (0, 0)),
        ],
        out_specs=pl.BlockSpec((G, H), lambda i: (0, 0)),
        out_shape=jax.ShapeDtypeStruct((G, H), jnp.float32),
        scratch_shapes=[
            pltpu.VMEM((G, H), jnp.float32),
            pltpu.VMEM((G, 1), jnp.float32),
        ],
    )(agg2, dego, batchp, Wg, bg, W1, b1, W2, b2)


def kernel(x, edge_index, batch, Wg, bg, W1, b1, W2, b2):
    e = edge_index.shape[1]
    src = edge_index[0]
    dst = edge_index[1]

    chunk_a = NS * K * SUPER * LANES
    e_pad_a = -(-e // chunk_a) * chunk_a
    rows_a = e_pad_a // LANES
    pad_a = e_pad_a - e
    srcp = jnp.concatenate([src, jnp.zeros((pad_a,), jnp.int32)])
    dstp = jnp.concatenate([dst, jnp.full((pad_a,), N, jnp.int32)])
    src2 = jnp.stack([2 * srcp, 2 * srcp + 1]).reshape(NC, rows_a, LANES)
    dstr = dstp.reshape(rows_a, LANES)

    chunk_b = NC * NS * KD * LANES
    e_pad_b = -(-e // chunk_b) * chunk_b
    pad_b = e_pad_b - e
    dstp_b = jnp.concatenate([dst, jnp.full((pad_b,), N, jnp.int32)])
    dst3 = dstp_b.reshape(NC, e_pad_b // (NC * LANES), LANES)

    xcat = x.reshape(2 * N, HALF)
    agg2, dego = _sc_aggregate(xcat, src2, dstr, dst3)

    batchp = jnp.concatenate(
        [batch, jnp.full((N_PAD - N,), G, jnp.int32)]).reshape(N_PAD, 1)
    return _tc_head(agg2, dego, batchp, Wg, bg.reshape(1, H),
                    W1, b1.reshape(1, H), W2, b2.reshape(1, H))


# speedup vs baseline: 1.7920x; 1.7920x over previous
import functools

import jax
import jax.numpy as jnp
from jax import lax
from jax.experimental import pallas as pl
from jax.experimental.pallas import tpu as pltpu
from jax.experimental.pallas import tpu_sc as plsc

N = 50000
D = 64
H = 64
G = 128
HALF = 32
NC = 2
NS = 16
LANES = 128
K = 4
KD = 8
DW = 8
N_PAD = 50048
ROWS_PER_TILE = N_PAD // NS
BLK = 3128
NBLK = N_PAD // BLK


def _feat_body(xcat, src2, dstr, z32, agg2,
               srcQ, dstQ, rows_v, acc_sh, isem0, isem1, gsem, ssem):
    c = lax.axis_index("c")
    s = lax.axis_index("s")
    rows_e_tile = src2.shape[1] // NS
    n_super = rows_e_tile // K
    base = s * rows_e_tile

    r0 = s * ROWS_PER_TILE
    pltpu.sync_copy(z32.at[pl.ds(r0, ROWS_PER_TILE)],
                    acc_sh.at[pl.ds(r0, ROWS_PER_TILE)])
    plsc.subcore_barrier()

    def fire_idx(g, slot, sem):
        row0 = base + g * K
        pltpu.async_copy(src2.at[c, pl.ds(row0, K)], srcQ.at[slot], sem)
        pltpu.async_copy(dstr.at[pl.ds(row0, K)], dstQ.at[slot], sem)

    def wait_idx(slot, sem):
        pltpu.make_async_copy(
            src2.at[c, pl.ds(base, K)], srcQ.at[slot], sem).wait()
        pltpu.make_async_copy(
            dstr.at[pl.ds(base, K)], dstQ.at[slot], sem).wait()

    def run_chunk(slot):
        descs = [pltpu.async_copy(
            xcat.at[srcQ.at[slot, j]],
            rows_v.at[pl.ds(j * LANES, LANES)], gsem) for j in range(K)]
        sdescs = []
        for j in range(K):
            descs[j].wait()
            sdescs.append(pltpu.async_copy(
                rows_v.at[pl.ds(j * LANES, LANES)],
                acc_sh.at[dstQ.at[slot, j]], ssem, add=True))
        return sdescs

    def wait_all(sdescs):
        for d_ in sdescs:
            d_.wait()

    fire_idx(0, 0, isem0)

    def super2(p, carry):
        g0 = 2 * p
        fire_idx(g0 + 1, 1, isem1)
        wait_idx(0, isem0)
        sd0 = run_chunk(0)
        wait_idx(1, isem1)
        wait_all(sd0)
        fire_idx(lax.rem(g0 + 2, n_super), 0, isem0)
        sd1 = run_chunk(1)
        wait_all(sd1)
        return carry

    lax.fori_loop(0, n_super // 2, super2, 0)
    wait_idx(0, isem0)
    plsc.subcore_barrier()

    pltpu.sync_copy(acc_sh.at[pl.ds(r0, ROWS_PER_TILE)],
                    agg2.at[c, pl.ds(r0, ROWS_PER_TILE)])


def _deg_body(dst3, zdeg, ones8, dego,
              dst_v, ones_v, deg_sh, dsem):
    c = lax.axis_index("c")
    s = lax.axis_index("s")
    rows_tile = dst3.shape[1] // NS
    n_chunks = rows_tile // KD

    pltpu.sync_copy(ones8, ones_v)
    r0 = s * ROWS_PER_TILE
    pltpu.sync_copy(zdeg.at[pl.ds(r0, ROWS_PER_TILE)],
                    deg_sh.at[pl.ds(r0, ROWS_PER_TILE)])
    plsc.subcore_barrier()

    def chunk(g, carry):
        row0 = s * rows_tile + g * KD
        pltpu.sync_copy(dst3.at[c, pl.ds(row0, KD)], dst_v)
        descs = []
        for j in range(KD):
            descs.append(pltpu.async_copy(
                ones_v, deg_sh.at[dst_v.at[j]], dsem, add=True))
        for d_ in descs:
            d_.wait()
        return carry

    lax.fori_loop(0, n_chunks, chunk, 0)
    plsc.subcore_barrier()

    pltpu.sync_copy(deg_sh.at[pl.ds(r0, ROWS_PER_TILE)],
                    dego.at[c, pl.ds(r0, ROWS_PER_TILE)])


def _sc_aggregate(xcat, src2, dstr, dst3):
    mesh = plsc.VectorSubcoreMesh(core_axis_name="c", subcore_axis_name="s")
    z32 = jnp.zeros((N_PAD, HALF), jnp.float32)
    feat = pl.kernel(
        _feat_body,
        out_type=jax.ShapeDtypeStruct((NC, N_PAD, HALF), jnp.float32),
        mesh=mesh,
        compiler_params=pltpu.CompilerParams(use_tc_tiling_on_sc=False),
        scratch_types=[
            pltpu.VMEM((2, K, LANES), jnp.int32),
            pltpu.VMEM((2, K, LANES), jnp.int32),
            pltpu.VMEM((K * LANES, HALF), jnp.float32),
            pltpu.VMEM_SHARED((N_PAD, HALF), jnp.float32),
            pltpu.SemaphoreType.DMA,
            pltpu.SemaphoreType.DMA,
            pltpu.SemaphoreType.DMA,
            pltpu.SemaphoreType.DMA,
        ],
    )
    agg2 = feat(xcat, src2, dstr, z32)

    zdeg = jnp.zeros((N_PAD, DW), jnp.float32)
    ones8 = jnp.ones((LANES, DW), jnp.float32)
    deg = pl.kernel(
        _deg_body,
        out_type=jax.ShapeDtypeStruct((NC, N_PAD, DW), jnp.float32),
        mesh=mesh,
        compiler_params=pltpu.CompilerParams(use_tc_tiling_on_sc=False),
        scratch_types=[
            pltpu.VMEM((KD, LANES), jnp.int32),
            pltpu.VMEM((LANES, DW), jnp.float32),
            pltpu.VMEM_SHARED((N_PAD, DW), jnp.float32),
            pltpu.SemaphoreType.DMA,
        ],
    )
    dego = deg(dst3, zdeg, ones8)
    return agg2, dego


def _tc_body(a2_ref, deg_ref, batch_ref, wg_ref, bg_ref, w1_ref, b1_ref,
             w2_ref, b2_ref, out_ref, g_acc, c_acc):
    i = pl.program_id(0)
    a = a2_ref[...]
    d = deg_ref[...]
    deg = d[0, :, 0:1] + d[1, :, 0:1]
    inv = 1.0 / jnp.maximum(deg, 1.0)
    n0 = a[0] * inv
    n1 = a[1] * inv
    h = n0 @ wg_ref[0:HALF, :] + n1 @ wg_ref[HALF:D, :] + bg_ref[...]
    h = jnp.maximum(h, 0.0)
    b_ids = batch_ref[...]
    oh = (b_ids == lax.broadcasted_iota(jnp.int32, (BLK, G), 1))
    oh = oh.astype(jnp.float32)
    g_part = lax.dot_general(oh, h, (((0,), (0,)), ((), ())))
    c_part = lax.dot_general(oh, jnp.ones((BLK, 1), jnp.float32),
                             (((0,), (0,)), ((), ())))

    @pl.when(i == 0)
    def _():
        g_acc[...] = jnp.zeros_like(g_acc)
        c_acc[...] = jnp.zeros_like(c_acc)

    g_acc[...] += g_part
    c_acc[...] += c_part

    @pl.when(i == NBLK - 1)
    def _():
        gm = g_acc[...] / jnp.maximum(c_acc[...], 1.0)
        t = jnp.maximum(gm @ w1_ref[...] + b1_ref[...], 0.0)
        out_ref[...] = t @ w2_ref[...] + b2_ref[...]


def _tc_head(agg2, dego, batchp, Wg, bg, W1, b1, W2, b2):
    return pl.pallas_call(
        _tc_body,
        grid=(NBLK,),
        in_specs=[
            pl.BlockSpec((NC, BLK, HALF), lambda i: (0, i, 0)),
            pl.BlockSpec((NC, BLK, DW), lambda i: (0, i, 0)),
            pl.BlockSpec((BLK, 1), lambda i: (i, 0)),
            pl.BlockSpec((D, H), lambda i: (0, 0)),
            pl.BlockSpec((1, H), lambda i: (0, 0)),
            pl.BlockSpec((H, H), lambda i: (0, 0)),
            pl.BlockSpec((1, H), lambda i: (0, 0)),
            pl.BlockSpec((H, H), lambda i: (0, 0)),
            pl.BlockSpec((1, H), lambda i: (0, 0)),
        ],
        out_specs=pl.BlockSpec((G, H), lambda i: (0, 0)),
        out_shape=jax.ShapeDtypeStruct((G, H), jnp.float32),
        scratch_shapes=[
            pltpu.VMEM((G, H), jnp.float32),
            pltpu.VMEM((G, 1), jnp.float32),
        ],
    )(agg2, dego, batchp, Wg, bg, W1, b1, W2, b2)


def kernel(x, edge_index, batch, Wg, bg, W1, b1, W2, b2):
    e = edge_index.shape[1]
    src = edge_index[0]
    dst = edge_index[1]

    chunk_a = 2 * NS * K * LANES
    e_pad_a = -(-e // chunk_a) * chunk_a
    rows_a = e_pad_a // LANES
    pad_a = e_pad_a - e
    srcp = jnp.concatenate([src, jnp.zeros((pad_a,), jnp.int32)])
    dstp = jnp.concatenate([dst, jnp.full((pad_a,), N, jnp.int32)])
    src2 = jnp.stack([2 * srcp, 2 * srcp + 1]).reshape(NC, rows_a, LANES)
    dstr = dstp.reshape(rows_a, LANES)

    chunk_b = NC * NS * KD * LANES
    e_pad_b = -(-e // chunk_b) * chunk_b
    pad_b = e_pad_b - e
    dstp_b = jnp.concatenate([dst, jnp.full((pad_b,), N, jnp.int32)])
    dst3 = dstp_b.reshape(NC, e_pad_b // (NC * LANES), LANES)

    xcat = x.reshape(2 * N, HALF)
    agg2, dego = _sc_aggregate(xcat, src2, dstr, dst3)

    batchp = jnp.concatenate(
        [batch, jnp.full((N_PAD - N,), G, jnp.int32)]).reshape(N_PAD, 1)
    return _tc_head(agg2, dego, batchp, Wg, bg.reshape(1, H),
                    W1, b1.reshape(1, H), W2, b2.reshape(1, H))
